# Initial kernel scaffold; baseline (speedup 1.0000x reference)
import jax
import jax.numpy as jnp
from jax.experimental import pallas as pl


def kernel(atom_feats, res_feats, edge_attr, edge_index, a2r_edge_index, res_edge_index, ca_idx, W_edge, W_atom_upd, W_a2r, W_res_msg, W_r2a, W_out):
    raise NotImplementedError("write your pallas kernel here")


# speedup vs baseline: 35.1621x; 35.1621x over previous
import functools

import jax
import jax.numpy as jnp
from jax import lax
from jax.experimental import pallas as pl
from jax.experimental.pallas import tpu as pltpu
from jax.experimental.pallas import tpu_sc as plsc

NA = 50000
NR = 6250
EA = 800000
EAR = 50000
ER = 187500
NS = 16
L = 16
CH = 80
ER_P = 2344 * CH
NCA = 6320
NRP = 6256

f32 = jnp.float32
_mesh = plsc.VectorSubcoreMesh(core_axis_name="c", subcore_axis_name="s")
_sc_params = pltpu.CompilerParams(use_tc_tiling_on_sc=False)


def _gate_body(ea_ref, w_ref, o_ref):
    o_ref[...] = jnp.maximum(
        jnp.dot(ea_ref[...], w_ref[...], preferred_element_type=f32), 0.0)


def _gate(edge_attr, W_edge):
    B = 2000
    return pl.pallas_call(
        _gate_body,
        grid=(EA // B,),
        in_specs=[pl.BlockSpec((B, 32), lambda i: (i, 0)),
                  pl.BlockSpec((32, 16), lambda i: (0, 0))],
        out_specs=pl.BlockSpec((B, 16), lambda i: (i, 0)),
        out_shape=jax.ShapeDtypeStruct((EA, 16), f32),
    )(edge_attr, W_edge)


def _upd_body(x_ref, g_ref, w_ref, o_ref):
    x = x_ref[0]
    g = g_ref[0]
    w = w_ref[...]
    a = jnp.dot(g[:, :16], w, preferred_element_type=f32)
    b = jnp.dot(g[:, 16:], w, preferred_element_type=f32)
    o_ref[0] = x + jnp.concatenate([a, b], axis=1)


def _upd(x_st, g_st, w):
    B = 2000
    return pl.pallas_call(
        _upd_body,
        grid=(2, NA // B),
        in_specs=[pl.BlockSpec((1, B, 32), lambda h, i: (h, i, 0)),
                  pl.BlockSpec((1, B, 32), lambda h, i: (h, i, 0)),
                  pl.BlockSpec((16, 16), lambda h, i: (0, 0))],
        out_specs=pl.BlockSpec((1, B, 32), lambda h, i: (h, i, 0)),
        out_shape=jax.ShapeDtypeStruct((2, NA, 32), f32),
    )(x_st, g_st, w)


def _res_body(ragg_ref, rf_ref, wa_ref, wm_ref, rhlo_ref, rhhi_ref, t_ref):
    wa = wa_ref[...]
    wm = wm_ref[...]
    rlo = ragg_ref[0:NR, :]
    rhi = ragg_ref[NRP:NRP + NR, :]
    rf = rf_ref[...]

    def fuse(rc, fc):
        return jnp.dot(jnp.concatenate([rc, fc], axis=1), wa,
                       preferred_element_type=f32)

    f0 = fuse(rlo[:, :16], rf[:, 0:32])
    f1 = fuse(rlo[:, 16:], rf[:, 32:64])
    f2 = fuse(rhi[:, :16], rf[:, 64:96])
    f3 = fuse(rhi[:, 16:], rf[:, 96:128])
    rhlo_ref[...] = jnp.concatenate([f0, f1], axis=1)
    rhhi_ref[...] = jnp.concatenate([f2, f3], axis=1)
    t = jnp.dot(f0, wm, preferred_element_type=f32)
    t_ref[0:NR, :] = t[:, :16]
    t_ref[NR:2 * NR, :] = t[:, 16:]


def _res(ragg, rf2d, W_a2r, W_res_msg):
    return pl.pallas_call(
        _res_body,
        out_shape=(jax.ShapeDtypeStruct((NR, 64), f32),
                   jax.ShapeDtypeStruct((NR, 64), f32),
                   jax.ShapeDtypeStruct((2 * NR, 16), f32)),
    )(ragg, rf2d, W_a2r, W_res_msg)


def _caimp_body(rhlo_ref, rhhi_ref, sagg_ref, ca_ref, wr_ref, o_ref):
    wr = wr_ref[...]
    rhlo = rhlo_ref[...]
    rhhi = rhhi_ref[...]
    sagg = jnp.concatenate([sagg_ref[0:NR, :], sagg_ref[NRP:NRP + NR, :]],
                           axis=1)
    s_new = rhlo[:, :32] + sagg
    calo = ca_ref[0:NR, :]
    cahi = ca_ref[NCA:NCA + NR, :]

    def imp(cac, rc):
        return jnp.dot(jnp.concatenate([cac, rc], axis=1), wr,
                       preferred_element_type=f32)

    i0 = imp(calo[:, :16], s_new)
    i1 = imp(calo[:, 16:], rhlo[:, 32:])
    i2 = imp(cahi[:, :16], rhhi[:, :32])
    i3 = imp(cahi[:, 16:], rhhi[:, 32:])
    o_ref[0:NR, :] = jnp.concatenate([i0, i1], axis=1)
    o_ref[NCA:NCA + NR, :] = jnp.concatenate([i2, i3], axis=1)


def _caimp(rh_lo, rh_hi, sagg, ca_st, W_r2a):
    return pl.pallas_call(
        _caimp_body,
        out_shape=jax.ShapeDtypeStruct((2 * NCA, 32), f32),
    )(rh_lo, rh_hi, sagg, ca_st, W_r2a)


def _final_body(xl_ref, xh_ref, gl_ref, gh_ref, w_ref, o_ref):
    w = w_ref[...]

    def half(x, g):
        a = jnp.dot(g[:, :16], w, preferred_element_type=f32)
        b = jnp.dot(g[:, 16:], w, preferred_element_type=f32)
        return x + jnp.concatenate([a, b], axis=1)

    o_ref[...] = jnp.concatenate([half(xl_ref[...], gl_ref[...]),
                                  half(xh_ref[...], gh_ref[...])], axis=1)


def _final(ah2, agg2, w):
    B = 2000
    lo = lambda i: (i, 0)
    hi = lambda i: (i + NA // B, 0)
    return pl.pallas_call(
        _final_body,
        grid=(NA // B,),
        in_specs=[pl.BlockSpec((B, 32), lo), pl.BlockSpec((B, 32), hi),
                  pl.BlockSpec((B, 32), lo), pl.BlockSpec((B, 32), hi),
                  pl.BlockSpec((16, 16), lambda i: (0, 0))],
        out_specs=pl.BlockSpec((B, 64), lo),
        out_shape=jax.ShapeDtypeStruct((NA, 64), f32),
    )(ah2, ah2, agg2, agg2, w)


def _zero_vmem(zbuf):
    nrows = zbuf.shape[0]

    @pl.loop(0, nrows)
    def _(e):
        zbuf[e, 0:16] = jnp.zeros((16,), f32)
        if zbuf.shape[1] == 32:
            zbuf[e, 16:32] = jnp.zeros((16,), f32)


def _zero_acc(sid, zbuf, acc, nchunks):
    _zero_vmem(zbuf)
    zr = zbuf.shape[0]

    @pl.loop(sid, nchunks, step=NS)
    def _(i):
        pltpu.sync_copy(zbuf, acc.at[pl.ds(i * zr, zr)])


def _edge_pass(cid, sid, tab, gate, src, dst, acc,
               sidx, didx, gatev, rows, sem, nchunks):
    bias = cid * NA

    @pl.loop(sid, nchunks, step=NS)
    def _(k):
        off = k * CH
        pltpu.sync_copy(src.at[pl.ds(off, CH)], sidx)
        pltpu.sync_copy(dst.at[pl.ds(off, CH)], didx)
        pltpu.sync_copy(gate.at[pl.ds(off, CH)], gatev)
        for j in range(CH // L):
            sidx[pl.ds(j * L, L)] = sidx[pl.ds(j * L, L)] + bias
        pltpu.async_copy(tab.at[sidx], rows, sem).wait()

        @pl.loop(0, CH)
        def _(e):
            g = gatev[e, 0:16]
            rows[e, 0:16] = rows[e, 0:16] * g
            rows[e, 16:32] = rows[e, 16:32] * g

        pltpu.sync_copy(rows, acc.at[didx], add=True)


def _acc_writeout(sid, acc, out, nchunks, rows, out_base):
    @pl.loop(sid, nchunks, step=NS)
    def _(i):
        pltpu.sync_copy(acc.at[pl.ds(i * rows, rows)],
                        out.at[pl.ds(out_base + i * rows, rows)])


def _sc_a2a_body(tab, gate, src, dst, out, sidx, didx, gatev, rows, zbuf,
                 acc, sem):
    cid = lax.axis_index("c")
    sid = lax.axis_index("s")
    _zero_acc(sid, zbuf, acc, 250)
    plsc.subcore_barrier()
    _edge_pass(cid, sid, tab, gate, src, dst, acc,
               sidx, didx, gatev, rows, sem, EA // CH)
    plsc.subcore_barrier()
    _acc_writeout(sid, acc, out, 250, 200, cid * NA)


_sc_a2a = pl.kernel(
    _sc_a2a_body,
    out_type=jax.ShapeDtypeStruct((2 * NA, 32), f32),
    mesh=_mesh,
    compiler_params=_sc_params,
    scratch_types=[pltpu.VMEM((CH,), jnp.int32), pltpu.VMEM((CH,), jnp.int32),
                   pltpu.VMEM((CH, 16), f32), pltpu.VMEM((CH, 32), f32),
                   pltpu.VMEM((200, 32), f32),
                   pltpu.VMEM_SHARED((NA, 32), f32),
                   pltpu.SemaphoreType.DMA],
)


def _sc_a2r_body(tab, asrc, adst, out, sidx, didx, rows, zbuf, acc, sem):
    cid = lax.axis_index("c")
    sid = lax.axis_index("s")
    _zero_acc(sid, zbuf, acc, 17)
    plsc.subcore_barrier()
    bias = cid * NA

    @pl.loop(sid, EAR // CH, step=NS)
    def _(k):
        off = k * CH
        pltpu.sync_copy(asrc.at[pl.ds(off, CH)], sidx)
        pltpu.sync_copy(adst.at[pl.ds(off, CH)], didx)
        for j in range(CH // L):
            sidx[pl.ds(j * L, L)] = sidx[pl.ds(j * L, L)] + bias
        pltpu.async_copy(tab.at[sidx], rows, sem).wait()
        pltpu.sync_copy(rows, acc.at[didx], add=True)

    plsc.subcore_barrier()
    _acc_writeout(sid, acc, out, 17, 368, cid * NRP)


_sc_a2r = pl.kernel(
    _sc_a2r_body,
    out_type=jax.ShapeDtypeStruct((2 * NRP, 32), f32),
    mesh=_mesh,
    compiler_params=_sc_params,
    scratch_types=[pltpu.VMEM((CH,), jnp.int32), pltpu.VMEM((CH,), jnp.int32),
                   pltpu.VMEM((CH, 32), f32), pltpu.VMEM((368, 32), f32),
                   pltpu.VMEM_SHARED((NRP, 32), f32),
                   pltpu.SemaphoreType.DMA],
)


def _sc_res_body(t_tab, rsrc, rdst, ah_tab, ca_g, sagg_out, ca_out,
                 sidx, didx, rows16, carows, zbuf, acc, sem):
    cid = lax.axis_index("c")
    sid = lax.axis_index("s")
    _zero_acc(sid, zbuf, acc, 17)
    plsc.subcore_barrier()

    @pl.loop(sid, NCA // CH, step=NS)
    def _(k):
        off = k * CH
        pltpu.sync_copy(ca_g.at[pl.ds(off, CH)], sidx)
        for j in range(CH // L):
            sidx[pl.ds(j * L, L)] = sidx[pl.ds(j * L, L)] + cid * NA
        pltpu.async_copy(ah_tab.at[sidx], carows, sem).wait()
        pltpu.sync_copy(carows, ca_out.at[pl.ds(cid * NCA + off, CH)])

    @pl.loop(sid, ER_P // CH, step=NS)
    def _(k):
        off = k * CH
        pltpu.sync_copy(rsrc.at[pl.ds(off, CH)], sidx)
        pltpu.sync_copy(rdst.at[pl.ds(off, CH)], didx)
        for j in range(CH // L):
            sidx[pl.ds(j * L, L)] = sidx[pl.ds(j * L, L)] + cid * NR

        pltpu.async_copy(t_tab.at[sidx], rows16, sem).wait()

        @pl.loop(0, CH)
        def _(e):
            rows16[e, 0:16] = jnp.maximum(rows16[e, 0:16], 0.0)

        pltpu.sync_copy(rows16, acc.at[didx], add=True)

    plsc.subcore_barrier()
    _acc_writeout(sid, acc, sagg_out, 17, 368, cid * NRP)


_sc_res = pl.kernel(
    _sc_res_body,
    out_type=(jax.ShapeDtypeStruct((2 * NRP, 16), f32),
              jax.ShapeDtypeStruct((2 * NCA, 32), f32)),
    mesh=_mesh,
    compiler_params=_sc_params,
    scratch_types=[pltpu.VMEM((CH,), jnp.int32), pltpu.VMEM((CH,), jnp.int32),
                   pltpu.VMEM((CH, 16), f32), pltpu.VMEM((CH, 32), f32),
                   pltpu.VMEM((368, 16), f32),
                   pltpu.VMEM_SHARED((NRP, 16), f32),
                   pltpu.SemaphoreType.DMA],
)


def _sc_h_body(ah, gate, src, dst, casc, caimp, ah2, agg2,
               sidx, didx, gatev, rows, zbuf, vbuf, acc, sem):
    cid = lax.axis_index("c")
    sid = lax.axis_index("s")
    _zero_acc(sid, zbuf, acc, 250)

    @pl.loop(sid, 250, step=NS)
    def _(i):
        r = cid * NA + i * 200
        pltpu.sync_copy(ah.at[pl.ds(r, 200)], vbuf)
        pltpu.sync_copy(vbuf, ah2.at[pl.ds(r, 200)])

    plsc.subcore_barrier()

    @pl.loop(sid, NCA // CH, step=NS)
    def _(k):
        off = k * CH
        pltpu.sync_copy(casc.at[pl.ds(off, CH)], sidx)
        for j in range(CH // L):
            v = sidx[pl.ds(j * L, L)]
            sidx[pl.ds(j * L, L)] = jnp.where(v < NA, v + cid * NA, 2 * NA)
        pltpu.sync_copy(caimp.at[pl.ds(cid * NCA + off, CH)], rows)
        pltpu.sync_copy(rows, ah2.at[sidx])

    plsc.subcore_barrier()
    _edge_pass(cid, sid, ah2, gate, src, dst, acc,
               sidx, didx, gatev, rows, sem, EA // CH)
    plsc.subcore_barrier()
    _acc_writeout(sid, acc, agg2, 250, 200, cid * NA)


_sc_h = pl.kernel(
    _sc_h_body,
    out_type=(jax.ShapeDtypeStruct((2 * NA + 8, 32), f32),
              jax.ShapeDtypeStruct((2 * NA, 32), f32)),
    mesh=_mesh,
    compiler_params=_sc_params,
    scratch_types=[pltpu.VMEM((CH,), jnp.int32), pltpu.VMEM((CH,), jnp.int32),
                   pltpu.VMEM((CH, 16), f32), pltpu.VMEM((CH, 32), f32),
                   pltpu.VMEM((200, 32), f32), pltpu.VMEM((200, 32), f32),
                   pltpu.VMEM_SHARED((NA, 32), f32),
                   pltpu.SemaphoreType.DMA],
)


def kernel(atom_feats, res_feats, edge_attr, edge_index, a2r_edge_index,
           res_edge_index, ca_idx, W_edge, W_atom_upd, W_a2r, W_res_msg,
           W_r2a, W_out):
    af = atom_feats.reshape(NA, 64)
    af_st = jnp.concatenate([af[:, :32], af[:, 32:]], axis=0)
    src = edge_index[0]
    dst = edge_index[1]
    asrc = a2r_edge_index[0]
    adst = a2r_edge_index[1]
    rsrc = jnp.pad(res_edge_index[0], (0, ER_P - ER))
    rdst = jnp.pad(res_edge_index[1], (0, ER_P - ER), constant_values=NR)
    r_ids = jnp.arange(NR, dtype=jnp.int32)
    win = jnp.full((NA,), -1, jnp.int32).at[ca_idx].max(r_ids)
    ca_sc = jnp.where(win[ca_idx] == r_ids, ca_idx, NA)
    ca_sc_p = jnp.pad(ca_sc, (0, NCA - NR), constant_values=NA)
    ca_g_p = jnp.pad(ca_idx, (0, NCA - NR))
    rf2d = res_feats.reshape(NR, 128)

    gate = _gate(edge_attr, W_edge)
    agg = _sc_a2a(af_st, gate, src, dst)
    ah_st = _upd(af_st.reshape(2, NA, 32), agg.reshape(2, NA, 32),
                 W_atom_upd).reshape(2 * NA, 32)
    ragg = _sc_a2r(ah_st, asrc, adst)
    rh_lo, rh_hi, t_st = _res(ragg, rf2d, W_a2r, W_res_msg)
    t_st_p = jnp.pad(t_st, ((0, 12), (0, 0)))
    sagg, ca_st = _sc_res(t_st_p, rsrc, rdst, ah_st, ca_g_p)
    ca_imp = _caimp(rh_lo, rh_hi, sagg, ca_st, W_r2a)
    ah2, agg2 = _sc_h(ah_st, gate, src, dst, ca_sc_p, ca_imp)
    out2d = _final(ah2, agg2, W_out)
    return out2d.reshape(NA, 4, 16)
